# baseline (speedup 1.0000x reference)
import functools

import jax
import jax.numpy as jnp
from jax import lax
from jax.experimental import pallas as pl
from jax.experimental.pallas import tpu as pltpu
from jax.experimental.pallas import tpu_sc as plsc

L = 16
NC = 2
NS = 16


@functools.lru_cache(maxsize=None)
def _build(n, k, f, n_up):
    fp = f + 128
    e_total = n * k
    lchunk = 2048
    rounds = -(-e_total // (NS * lchunk))
    ep_tile = rounds * lchunk
    e_pad = NS * ep_tile
    half = (n // 2 + L - 1) // L * L
    maskn = (n + 1 + L - 1) // L * L
    b = 64
    rh = 48
    nh = 4
    ro_slots = nh * rh
    sel_pad = NS * ro_slots
    out_rows = sel_pad + L
    out_trash = out_rows - 1
    spt = (sel_pad // NS + 15) // 8 * 8
    sc_rows = NS * spt
    obuf_sz = lchunk + 10 * L

    def body(feat, nidxf, wf, sel, out,
             acc, ldst, lsrc, lw, csh,
             mask, selv, nidx_c, w_c, dst_e, src_e, w_e, cntv, cntbuf,
             bdst, bsrc, bw, odst, osrc, ow, srcbuf, gbuf, accT,
             jl, ll, orows, sem, sem2):
        cid = lax.axis_index("c")
        sid = lax.axis_index("s")
        base = cid * half
        end = base + jnp.where(cid == 0, half, n - half)
        cbase = cid * sc_rows
        own0 = sid * spt
        iota = lax.iota(jnp.int32, L)
        zero16 = jnp.zeros((L,), jnp.float32)
        e0vec = jnp.where(iota == 0, 1.0, 0.0).astype(jnp.float32)
        sh5 = jnp.full((L,), 5, jnp.int32)
        one16f = jnp.full((L,), 1.0, jnp.float32)

        def zrow(r, _):
            for j in range(fp // L):
                accT[r, pl.ds(j * L, L)] = zero16
            return 0
        lax.fori_loop(0, spt, zrow, 0)

        def zm(g, _):
            mask[pl.ds(g * L, L)] = jnp.zeros((L,), jnp.int32)
            return 0
        lax.fori_loop(0, maskn // L, zm, 0)
        pltpu.sync_copy(sel, selv)
        def bm(g, _):
            nd = selv[pl.ds(g * L, L)]
            inr = (nd >= base) & (nd < end)
            jval = g * L + iota + 1
            plsc.store_scatter(mask, [nd], jval, mask=inr)
            return 0
        lax.fori_loop(0, sel_pad // L, bm, 0)

        def process_owned(nloc):
            a16 = nloc & jnp.int32(~(L - 1))
            nlocs = jnp.full((L,), 1, jnp.int32) * nloc
            zi = jnp.zeros((L,), jnp.int32)
            zf = jnp.zeros((L,), jnp.float32)
            for t in range(5):
                p = a16 + t * L
                mg = (p + iota) >= nlocs
                odst[pl.ds(p, L)] = jnp.where(mg, zi, odst[pl.ds(p, L)])
                osrc[pl.ds(p, L)] = jnp.where(mg, zi, osrc[pl.ds(p, L)])
                ow[pl.ds(p, L)] = jnp.where(mg, zf, ow[pl.ds(p, L)])
            nb = lax.shift_right_logical(nloc + (b - 1), 6)
            def batch_body(bi, _):
                bb = bi * b
                for q in range(b // L):
                    srcbuf[pl.ds(q * L, L)] = osrc[pl.ds(bb + q * L, L)]
                pltpu.async_copy(feat.at[srcbuf], gbuf, sem).wait()
                def edge_body(e, _):
                    row = odst[pl.ds(bb + e, L)][0]
                    ws = one16f * ow[pl.ds(bb + e, L)][0]
                    for j in range(f // L):
                        plsc.addupdate(accT.at[row, pl.ds(j * L, L)],
                                       gbuf[e, pl.ds(j * L, L)] * ws)
                    plsc.addupdate(accT.at[row, pl.ds(f, L)], ws * e0vec)
                    return 0
                lax.fori_loop(0, b, edge_body, 0)
                return 0
            lax.fori_loop(0, nb, batch_body, 0)

        def round_body(r, _):
            e0 = sid * ep_tile + r * lchunk
            pltpu.sync_copy(nidxf.at[pl.ds(e0, lchunk)], nidx_c)
            pltpu.sync_copy(wf.at[pl.ds(e0, lchunk)], w_c)
            def scan_g(g, off):
                nd = nidx_c[pl.ds(g * L, L)]
                wv = w_c[pl.ds(g * L, L)]
                m = plsc.load_gather(mask, [nd])
                keep = m > 0
                src = lax.shift_right_logical(e0 + g * L + iota, sh5)
                plsc.store_compressed(dst_e.at[pl.ds(off, L)], m - 1, mask=keep)
                plsc.store_compressed(src_e.at[pl.ds(off, L)], src, mask=keep)
                plsc.store_compressed(w_e.at[pl.ds(off, L)], wv, mask=keep)
                return off + jnp.max(plsc.all_reduce_population_count(keep))
            cnt = lax.fori_loop(0, lchunk // L, scan_g, jnp.int32(0))
            cntv[pl.ds(0, L)] = jnp.full((L,), 1, jnp.int32) * cnt
            pltpu.sync_copy(cntv, csh.at[pl.ds(sid * 128, L)])
            nblk = lax.shift_right_logical(cnt + 255, 8)
            def pub(bk, _):
                o = bk * 256
                pltpu.sync_copy(dst_e.at[pl.ds(o, 256)],
                                ldst.at[pl.ds(sid * lchunk + o, 256)])
                pltpu.sync_copy(src_e.at[pl.ds(o, 256)],
                                lsrc.at[pl.ds(sid * lchunk + o, 256)])
                pltpu.sync_copy(w_e.at[pl.ds(o, 256)],
                                lw.at[pl.ds(sid * lchunk + o, 256)])
                return 0
            lax.fori_loop(0, nblk, pub, 0)
            plsc.subcore_barrier()

            pltpu.sync_copy(csh, cntbuf)
            def per_scanner(t, _):
                cnt_t = cntbuf[pl.ds(t * 128, L)][0]
                nblk_t = lax.shift_right_logical(cnt_t + 255, 8)
                def pull(bk, _):
                    o = bk * 256
                    pltpu.sync_copy(ldst.at[pl.ds(t * lchunk + o, 256)],
                                    bdst.at[pl.ds(o, 256)])
                    pltpu.sync_copy(lsrc.at[pl.ds(t * lchunk + o, 256)],
                                    bsrc.at[pl.ds(o, 256)])
                    pltpu.sync_copy(lw.at[pl.ds(t * lchunk + o, 256)],
                                    bw.at[pl.ds(o, 256)])
                    return 0
                lax.fori_loop(0, nblk_t, pull, 0)
                cnts = jnp.full((L,), 1, jnp.int32) * cnt_t
                ng = lax.shift_right_logical(cnt_t + (L - 1), 4)
                def own_g(g, off):
                    s = bdst[pl.ds(g * L, L)]
                    valid = ((g * L + iota) < cnts) & (s >= own0) \
                        & (s < own0 + spt)
                    plsc.store_compressed(odst.at[pl.ds(off, L)], s - own0,
                                          mask=valid)
                    plsc.store_compressed(osrc.at[pl.ds(off, L)],
                                          bsrc[pl.ds(g * L, L)], mask=valid)
                    plsc.store_compressed(ow.at[pl.ds(off, L)],
                                          bw[pl.ds(g * L, L)], mask=valid)
                    return off + jnp.max(
                        plsc.all_reduce_population_count(valid))
                nloc = lax.fori_loop(0, ng, own_g, jnp.int32(0))
                process_owned(nloc)
                return 0
            lax.fori_loop(0, NS, per_scanner, 0)
            plsc.subcore_barrier()
            return 0
        lax.fori_loop(0, rounds, round_body, 0)

        pltpu.sync_copy(accT, acc.at[pl.ds(cbase + own0, spt)])
        plsc.subcore_barrier()

        to16 = jnp.full((L,), out_trash, jnp.int32)
        tr16 = jnp.full((L,), sc_rows - 1, jnp.int32) + cbase
        pltpu.sync_copy(sel.at[pl.ds(sid * ro_slots, ro_slots)],
                        selv.at[pl.ds(0, ro_slots)])
        def ro_pre(g, _):
            dst_e[pl.ds(g * L, L)] = to16
            src_e[pl.ds(g * L, L)] = tr16
            return 0
        lax.fori_loop(0, ro_slots // L, ro_pre, 0)
        jb = sid * ro_slots
        def ro_scan(g, off):
            nd = selv[pl.ds(g * L, L)]
            inr = (nd >= base) & (nd < end)
            jv = jb + g * L + iota
            lv = plsc.load_gather(mask, [jnp.maximum(nd, 0)]) - 1 + cbase
            plsc.store_compressed(dst_e.at[pl.ds(off, L)], jv, mask=inr)
            plsc.store_compressed(src_e.at[pl.ds(off, L)], lv, mask=inr)
            return off + jnp.max(plsc.all_reduce_population_count(inr))
        lax.fori_loop(0, ro_slots // L, ro_scan, jnp.int32(0))

        for h in range(nh):
            for g in range(rh // L):
                jl[pl.ds(g * L, L)] = dst_e[pl.ds(h * rh + g * L, L)]
                ll[pl.ds(g * L, L)] = src_e[pl.ds(h * rh + g * L, L)]
            pltpu.async_copy(acc.at[ll], orows, sem2).wait()
            def ro_row(r, _):
                dv = orows[r, pl.ds(f, L)]
                dens = one16f * dv[0]
                inv = jnp.where(dens == 0.0, 0.0, 1.0 / dens)
                for j in range(f // L):
                    orows[r, pl.ds(j * L, L)] = orows[r, pl.ds(j * L, L)] * inv
                return 0
            lax.fori_loop(0, rh, ro_row, 0)
            pltpu.sync_copy(orows, out.at[jl])

    mesh = plsc.VectorSubcoreMesh(core_axis_name="c", subcore_axis_name="s")
    kern = pl.kernel(
        body,
        out_type=jax.ShapeDtypeStruct((out_rows, fp), jnp.float32),
        mesh=mesh,
        compiler_params=pltpu.CompilerParams(needs_layout_passes=False),
        scratch_types=[
            pltpu.HBM((NC * sc_rows, fp), jnp.float32),
            pltpu.VMEM_SHARED((NS * lchunk,), jnp.int32),
            pltpu.VMEM_SHARED((NS * lchunk,), jnp.int32),
            pltpu.VMEM_SHARED((NS * lchunk,), jnp.float32),
            pltpu.VMEM_SHARED((NS * 128,), jnp.int32),
            pltpu.VMEM((maskn,), jnp.int32),
            pltpu.VMEM((sel_pad,), jnp.int32),
            pltpu.VMEM((lchunk,), jnp.int32),
            pltpu.VMEM((lchunk,), jnp.float32),
            pltpu.VMEM((lchunk,), jnp.int32),
            pltpu.VMEM((lchunk,), jnp.int32),
            pltpu.VMEM((lchunk,), jnp.float32),
            pltpu.VMEM((L,), jnp.int32),
            pltpu.VMEM((NS * 128,), jnp.int32),
            pltpu.VMEM((lchunk,), jnp.int32),
            pltpu.VMEM((lchunk,), jnp.int32),
            pltpu.VMEM((lchunk,), jnp.float32),
            pltpu.VMEM((obuf_sz,), jnp.int32),
            pltpu.VMEM((obuf_sz,), jnp.int32),
            pltpu.VMEM((obuf_sz,), jnp.float32),
            pltpu.VMEM((b,), jnp.int32),
            pltpu.VMEM((b, f), jnp.float32),
            pltpu.VMEM((spt, fp), jnp.float32),
            pltpu.VMEM((rh,), jnp.int32),
            pltpu.VMEM((rh,), jnp.int32),
            pltpu.VMEM((rh, fp), jnp.float32),
            pltpu.SemaphoreType.DMA,
            pltpu.SemaphoreType.DMA,
        ],
    )
    return kern, e_pad, sel_pad


def kernel(features, weights_down, nidx_down, sel_idx_up):
    n, f = features.shape
    k = weights_down.shape[1]
    n_up = sel_idx_up.shape[0]
    kern, e_pad, sel_pad = _build(n, k, f, n_up)
    e_total = n * k
    nidxf = jnp.concatenate(
        [nidx_down.reshape(-1),
         jnp.full((e_pad - e_total,), n, jnp.int32)])
    wf = jnp.concatenate(
        [weights_down.reshape(-1),
         jnp.zeros((e_pad - e_total,), jnp.float32)])
    selp = jnp.concatenate(
        [sel_idx_up[:, 0].astype(jnp.int32),
         jnp.full((sel_pad - n_up,), -1, jnp.int32)])
    out = kern(features, nidxf, wf, selp)
    return out[:n_up, :f]


# speedup vs baseline: 12.8501x; 12.8501x over previous
import functools

import jax
import jax.numpy as jnp
from jax import lax
from jax.experimental import pallas as pl
from jax.experimental.pallas import tpu as pltpu
from jax.experimental.pallas import tpu_sc as plsc

L = 16
NC = 2
NS = 16


@functools.lru_cache(maxsize=None)
def _build(n, k, f, n_up):
    fp = f + 128
    e_total = n * k
    lchunk = 2048
    rounds = -(-e_total // (NS * lchunk))
    ep_tile = rounds * lchunk
    e_pad = NS * ep_tile
    half = (n // 2 + L - 1) // L * L
    maskn = (n + 1 + L - 1) // L * L
    b = 32
    rh = 32
    nh = 6
    ro_slots = nh * rh
    sel_pad = NS * ro_slots
    out_rows = sel_pad + L
    out_trash = out_rows - 1
    spt = (sel_pad // NS + 15) // 8 * 8
    sc_rows = NS * spt
    ebuf = 3 * lchunk
    osz = lchunk + b + 3 * L

    def body(feat, nidxf, wf, sel, out,
             acc, lsh, csh,
             mask, selv, nidx_c, w_c, edata, cntv, cntbuf, bdata,
             odst, osrc, ow, srcbuf, gbuf, accT, jl, ll, orows, sem, sem2):
        cid = lax.axis_index("c")
        sid = lax.axis_index("s")
        base = cid * half
        end = base + jnp.where(cid == 0, half, n - half)
        cbase = cid * sc_rows
        own0 = sid * spt
        iota = lax.iota(jnp.int32, L)
        zero16 = jnp.zeros((L,), jnp.float32)
        e0vec = jnp.where(iota == 0, 1.0, 0.0).astype(jnp.float32)
        sh5 = jnp.full((L,), 5, jnp.int32)
        one16f = jnp.full((L,), 1.0, jnp.float32)
        one16i = jnp.full((L,), 1, jnp.int32)

        def zrow(r, _):
            for j in range(fp // L):
                accT[r, pl.ds(j * L, L)] = zero16
            return 0
        lax.fori_loop(0, spt, zrow, 0)

        def zm(g, _):
            mask[pl.ds(g * L, L)] = jnp.zeros((L,), jnp.int32)
            return 0
        lax.fori_loop(0, maskn // L, zm, 0)
        pltpu.sync_copy(sel, selv)
        def bm(g, _):
            nd = selv[pl.ds(g * L, L)]
            inr = (nd >= base) & (nd < end)
            jval = g * L + iota + 1
            plsc.store_scatter(mask, [nd], jval, mask=inr)
            return 0
        lax.fori_loop(0, sel_pad // L, bm, 0)

        def run_batches(off):
            nb = lax.shift_right_logical(off, 5)
            def batch_body(bi, _):
                bb = bi * b
                for q in range(b // L):
                    srcbuf[pl.ds(q * L, L)] = osrc[pl.ds(bb + q * L, L)]
                pltpu.async_copy(feat.at[srcbuf], gbuf, sem).wait()
                def edge_body(e, _):
                    row = odst[pl.ds(bb + e, L)][0]
                    ws = one16f * ow[pl.ds(bb + e, L)][0]
                    for j in range(f // L):
                        plsc.addupdate(accT.at[row, pl.ds(j * L, L)],
                                       gbuf[e, pl.ds(j * L, L)] * ws)
                    plsc.addupdate(accT.at[row, pl.ds(f, L)], ws * e0vec)
                    return 0
                lax.fori_loop(0, b, edge_body, 0)
                return 0
            lax.fori_loop(0, nb, batch_body, 0)
            tb = nb * b
            for t in range(b // L):
                odst[pl.ds(t * L, L)] = odst[pl.ds(tb + t * L, L)]
                osrc[pl.ds(t * L, L)] = osrc[pl.ds(tb + t * L, L)]
                ow[pl.ds(t * L, L)] = ow[pl.ds(tb + t * L, L)]
            return off - tb

        def round_body(r, off):
            e0 = sid * ep_tile + r * lchunk
            pltpu.sync_copy(nidxf.at[pl.ds(e0, lchunk)], nidx_c)
            pltpu.sync_copy(wf.at[pl.ds(e0, lchunk)], w_c)
            def scan_g(g, sc_off):
                nd = nidx_c[pl.ds(g * L, L)]
                wv = w_c[pl.ds(g * L, L)]
                m = plsc.load_gather(mask, [nd])
                keep = m > 0
                src = lax.shift_right_logical(e0 + g * L + iota, sh5)
                plsc.store_compressed(edata.at[pl.ds(sc_off, L)], m - 1,
                                      mask=keep)
                plsc.store_compressed(edata.at[pl.ds(lchunk + sc_off, L)],
                                      src, mask=keep)
                plsc.store_compressed(edata.at[pl.ds(2 * lchunk + sc_off, L)],
                                      plsc.bitcast(wv, jnp.int32), mask=keep)
                return sc_off + jnp.max(plsc.all_reduce_population_count(keep))
            cnt = lax.fori_loop(0, lchunk // L, scan_g, jnp.int32(0))
            cntv[pl.ds(0, L)] = one16i * cnt
            pltpu.sync_copy(cntv, csh.at[pl.ds(sid * 128, L)])
            pltpu.sync_copy(edata, lsh.at[pl.ds(sid * ebuf, ebuf)])
            plsc.subcore_barrier()

            pltpu.sync_copy(csh, cntbuf)
            def per_scanner(t, off):
                cnt_t = cntbuf[pl.ds(t * 128, L)][0]
                pltpu.sync_copy(lsh.at[pl.ds(t * ebuf, ebuf)], bdata)
                cnts = one16i * cnt_t
                ng = lax.shift_right_logical(cnt_t + (L - 1), 4)
                def own_g(g, o):
                    s = bdata[pl.ds(g * L, L)]
                    valid = ((g * L + iota) < cnts) & (s >= own0) \
                        & (s < own0 + spt)
                    plsc.store_compressed(odst.at[pl.ds(o, L)], s - own0,
                                          mask=valid)
                    plsc.store_compressed(
                        osrc.at[pl.ds(o, L)],
                        bdata[pl.ds(lchunk + g * L, L)], mask=valid)
                    plsc.store_compressed(
                        ow.at[pl.ds(o, L)],
                        plsc.bitcast(bdata[pl.ds(2 * lchunk + g * L, L)],
                                     jnp.float32), mask=valid)
                    return o + jnp.max(plsc.all_reduce_population_count(valid))
                off = lax.fori_loop(0, ng, own_g, off)
                return run_batches(off)
            off = lax.fori_loop(0, NS, per_scanner, off)
            plsc.subcore_barrier()
            return off
        off = lax.fori_loop(0, rounds, round_body, jnp.int32(0))

        a16 = off & jnp.int32(~(L - 1))
        offs = one16i * off
        zi = jnp.zeros((L,), jnp.int32)
        zf = jnp.zeros((L,), jnp.float32)
        for t in range(b // L + 1):
            p = a16 + t * L
            mg = (p + iota) >= offs
            odst[pl.ds(p, L)] = jnp.where(mg, zi, odst[pl.ds(p, L)])
            osrc[pl.ds(p, L)] = jnp.where(mg, zi, osrc[pl.ds(p, L)])
            ow[pl.ds(p, L)] = jnp.where(mg, zf, ow[pl.ds(p, L)])
        run_batches((off + b - 1) & jnp.int32(~(b - 1)))

        pltpu.sync_copy(accT, acc.at[pl.ds(cbase + own0, spt)])
        plsc.subcore_barrier()

        to16 = jnp.full((L,), out_trash, jnp.int32)
        tr16 = jnp.full((L,), sc_rows - 1, jnp.int32) + cbase
        pltpu.sync_copy(sel.at[pl.ds(sid * ro_slots, ro_slots)],
                        selv.at[pl.ds(0, ro_slots)])
        def ro_pre(g, _):
            edata[pl.ds(g * L, L)] = to16
            edata[pl.ds(lchunk + g * L, L)] = tr16
            return 0
        lax.fori_loop(0, ro_slots // L, ro_pre, 0)
        jb = sid * ro_slots
        def ro_scan(g, o):
            nd = selv[pl.ds(g * L, L)]
            inr = (nd >= base) & (nd < end)
            jv = jb + g * L + iota
            lv = plsc.load_gather(mask, [jnp.maximum(nd, 0)]) - 1 + cbase
            plsc.store_compressed(edata.at[pl.ds(o, L)], jv, mask=inr)
            plsc.store_compressed(edata.at[pl.ds(lchunk + o, L)], lv,
                                  mask=inr)
            return o + jnp.max(plsc.all_reduce_population_count(inr))
        lax.fori_loop(0, ro_slots // L, ro_scan, jnp.int32(0))

        for h in range(nh):
            for g in range(rh // L):
                jl[pl.ds(g * L, L)] = edata[pl.ds(h * rh + g * L, L)]
                ll[pl.ds(g * L, L)] = edata[pl.ds(lchunk + h * rh + g * L, L)]
            pltpu.async_copy(acc.at[ll], orows, sem2).wait()
            def ro_row(rr, _):
                dv = orows[rr, pl.ds(f, L)]
                dens = one16f * dv[0]
                inv = jnp.where(dens == 0.0, 0.0, 1.0 / dens)
                for j in range(f // L):
                    orows[rr, pl.ds(j * L, L)] = \
                        orows[rr, pl.ds(j * L, L)] * inv
                return 0
            lax.fori_loop(0, rh, ro_row, 0)
            pltpu.sync_copy(orows, out.at[jl])

    mesh = plsc.VectorSubcoreMesh(core_axis_name="c", subcore_axis_name="s")
    kern = pl.kernel(
        body,
        out_type=jax.ShapeDtypeStruct((out_rows, fp), jnp.float32),
        mesh=mesh,
        compiler_params=pltpu.CompilerParams(needs_layout_passes=False),
        scratch_types=[
            pltpu.HBM((NC * sc_rows, fp), jnp.float32),
            pltpu.VMEM_SHARED((NS * 3 * lchunk,), jnp.int32),
            pltpu.VMEM_SHARED((NS * 128,), jnp.int32),
            pltpu.VMEM((maskn,), jnp.int32),
            pltpu.VMEM((sel_pad,), jnp.int32),
            pltpu.VMEM((lchunk,), jnp.int32),
            pltpu.VMEM((lchunk,), jnp.float32),
            pltpu.VMEM((3 * lchunk,), jnp.int32),
            pltpu.VMEM((L,), jnp.int32),
            pltpu.VMEM((NS * 128,), jnp.int32),
            pltpu.VMEM((3 * lchunk,), jnp.int32),
            pltpu.VMEM((lchunk + b + 3 * L,), jnp.int32),
            pltpu.VMEM((lchunk + b + 3 * L,), jnp.int32),
            pltpu.VMEM((lchunk + b + 3 * L,), jnp.float32),
            pltpu.VMEM((b,), jnp.int32),
            pltpu.VMEM((b, f), jnp.float32),
            pltpu.VMEM((spt, fp), jnp.float32),
            pltpu.VMEM((rh,), jnp.int32),
            pltpu.VMEM((rh,), jnp.int32),
            pltpu.VMEM((rh, fp), jnp.float32),
            pltpu.SemaphoreType.DMA,
            pltpu.SemaphoreType.DMA,
        ],
    )
    return kern, e_pad, sel_pad


def kernel(features, weights_down, nidx_down, sel_idx_up):
    n, f = features.shape
    k = weights_down.shape[1]
    n_up = sel_idx_up.shape[0]
    kern, e_pad, sel_pad = _build(n, k, f, n_up)
    e_total = n * k
    nidxf = jnp.concatenate(
        [nidx_down.reshape(-1),
         jnp.full((e_pad - e_total,), n, jnp.int32)])
    wf = jnp.concatenate(
        [weights_down.reshape(-1),
         jnp.zeros((e_pad - e_total,), jnp.float32)])
    selp = jnp.concatenate(
        [sel_idx_up[:, 0].astype(jnp.int32),
         jnp.full((sel_pad - n_up,), -1, jnp.int32)])
    out = kern(features, nidxf, wf, selp)
    return out[:n_up, :f]


# speedup vs baseline: 13.2429x; 1.0306x over previous
import functools

import jax
import jax.numpy as jnp
from jax import lax
from jax.experimental import pallas as pl
from jax.experimental.pallas import tpu as pltpu
from jax.experimental.pallas import tpu_sc as plsc

L = 16
NC = 2
NS = 16


@functools.lru_cache(maxsize=None)
def _build(n, k, f, n_up):
    fp = f + 128
    e_total = n * k
    lchunk = 2048
    rounds = -(-e_total // (NS * lchunk))
    ep_tile = rounds * lchunk
    e_pad = NS * ep_tile
    half = (n // 2 + L - 1) // L * L
    maskn = (n + 1 + L - 1) // L * L
    b = 64
    rh = 32
    nh = 6
    ro_slots = nh * rh
    sel_pad = NS * ro_slots
    out_rows = sel_pad + L
    out_trash = out_rows - 1
    spt = (sel_pad // NS + 15) // 8 * 8
    sc_rows = NS * spt
    ebuf = L + 3 * lchunk
    osz = lchunk + b + 3 * L

    def body(feat, nidxf, wf, sel, out,
             acc, lsh,
             mask, selv, nidx_c, w_c, edata, bdata,
             odst, osrc, ow, srcbuf, gbuf, accT, jl, ll, orows, sem, sem2):
        cid = lax.axis_index("c")
        sid = lax.axis_index("s")
        base = cid * half
        end = base + jnp.where(cid == 0, half, n - half)
        cbase = cid * sc_rows
        own0 = sid * spt
        iota = lax.iota(jnp.int32, L)
        zero16 = jnp.zeros((L,), jnp.float32)
        e0vec = jnp.where(iota == 0, 1.0, 0.0).astype(jnp.float32)
        sh5 = jnp.full((L,), 5, jnp.int32)
        one16f = jnp.full((L,), 1.0, jnp.float32)
        one16i = jnp.full((L,), 1, jnp.int32)

        def zrow(r, _):
            for j in range(fp // L):
                accT[r, pl.ds(j * L, L)] = zero16
            return 0
        lax.fori_loop(0, spt, zrow, 0)

        def zm(g, _):
            mask[pl.ds(g * L, L)] = jnp.zeros((L,), jnp.int32)
            return 0
        lax.fori_loop(0, maskn // L, zm, 0)
        pltpu.sync_copy(sel, selv)
        def bm(g, _):
            nd = selv[pl.ds(g * L, L)]
            inr = (nd >= base) & (nd < end)
            jval = g * L + iota + 1
            plsc.store_scatter(mask, [nd], jval, mask=inr)
            return 0
        lax.fori_loop(0, sel_pad // L, bm, 0)

        def run_batches(off):
            nb = lax.shift_right_logical(off, 6)
            def batch_body(bi, _):
                bb = bi * b
                for q in range(b // L):
                    srcbuf[pl.ds(q * L, L)] = osrc[pl.ds(bb + q * L, L)]
                pltpu.async_copy(feat.at[srcbuf], gbuf, sem).wait()
                def edge_body(e, _):
                    row = odst[pl.ds(bb + e, L)][0]
                    ws = one16f * ow[pl.ds(bb + e, L)][0]
                    for j in range(f // L):
                        plsc.addupdate(accT.at[row, pl.ds(j * L, L)],
                                       gbuf[e, pl.ds(j * L, L)] * ws)
                    plsc.addupdate(accT.at[row, pl.ds(f, L)], ws * e0vec)
                    return 0
                lax.fori_loop(0, b, edge_body, 0)
                return 0
            lax.fori_loop(0, nb, batch_body, 0)
            tb = nb * b
            for t in range(b // L):
                odst[pl.ds(t * L, L)] = odst[pl.ds(tb + t * L, L)]
                osrc[pl.ds(t * L, L)] = osrc[pl.ds(tb + t * L, L)]
                ow[pl.ds(t * L, L)] = ow[pl.ds(tb + t * L, L)]
            return off - tb

        def round_body(r, off):
            e0 = sid * ep_tile + r * lchunk
            pltpu.sync_copy(nidxf.at[pl.ds(e0, lchunk)], nidx_c)
            pltpu.sync_copy(wf.at[pl.ds(e0, lchunk)], w_c)
            def scan_g(g, sc_off):
                nd = nidx_c[pl.ds(g * L, L)]
                wv = w_c[pl.ds(g * L, L)]
                m = plsc.load_gather(mask, [nd])
                keep = m > 0
                src = lax.shift_right_logical(e0 + g * L + iota, sh5)
                plsc.store_compressed(edata.at[pl.ds(L + sc_off, L)], m - 1,
                                      mask=keep)
                plsc.store_compressed(edata.at[pl.ds(L + lchunk + sc_off, L)],
                                      src, mask=keep)
                plsc.store_compressed(
                    edata.at[pl.ds(L + 2 * lchunk + sc_off, L)],
                    plsc.bitcast(wv, jnp.int32), mask=keep)
                return sc_off + jnp.max(plsc.all_reduce_population_count(keep))
            cnt = lax.fori_loop(0, lchunk // L, scan_g, jnp.int32(0))
            edata[pl.ds(0, L)] = one16i * cnt
            pltpu.sync_copy(edata, lsh.at[pl.ds(sid * ebuf, ebuf)])
            plsc.subcore_barrier()

            def per_pair(t, off):
                pltpu.sync_copy(lsh.at[pl.ds(t * 2 * ebuf, 2 * ebuf)], bdata)
                for sub in (0, ebuf):
                    cnt_t = bdata[pl.ds(sub, L)][0]
                    cnts = one16i * cnt_t
                    ng = lax.shift_right_logical(cnt_t + (L - 1), 4)
                    def own_g(g, o):
                        s = bdata[pl.ds(sub + L + g * L, L)]
                        valid = ((g * L + iota) < cnts) & (s >= own0) \
                            & (s < own0 + spt)
                        plsc.store_compressed(odst.at[pl.ds(o, L)], s - own0,
                                              mask=valid)
                        plsc.store_compressed(
                            osrc.at[pl.ds(o, L)],
                            bdata[pl.ds(sub + L + lchunk + g * L, L)],
                            mask=valid)
                        plsc.store_compressed(
                            ow.at[pl.ds(o, L)],
                            plsc.bitcast(
                                bdata[pl.ds(sub + L + 2 * lchunk + g * L, L)],
                                jnp.float32), mask=valid)
                        return o + jnp.max(
                            plsc.all_reduce_population_count(valid))
                    off = lax.fori_loop(0, ng, own_g, off)
                    off = run_batches(off)
                return off
            off = lax.fori_loop(0, NS // 2, per_pair, off)
            plsc.subcore_barrier()
            return off
        off = lax.fori_loop(0, rounds, round_body, jnp.int32(0))

        a16 = off & jnp.int32(~(L - 1))
        offs = one16i * off
        zi = jnp.zeros((L,), jnp.int32)
        zf = jnp.zeros((L,), jnp.float32)
        for t in range(b // L + 1):
            p = a16 + t * L
            mg = (p + iota) >= offs
            odst[pl.ds(p, L)] = jnp.where(mg, zi, odst[pl.ds(p, L)])
            osrc[pl.ds(p, L)] = jnp.where(mg, zi, osrc[pl.ds(p, L)])
            ow[pl.ds(p, L)] = jnp.where(mg, zf, ow[pl.ds(p, L)])
        run_batches((off + b - 1) & jnp.int32(~(b - 1)))

        pltpu.sync_copy(accT, acc.at[pl.ds(cbase + own0, spt)])
        plsc.subcore_barrier()

        to16 = jnp.full((L,), out_trash, jnp.int32)
        tr16 = jnp.full((L,), sc_rows - 1, jnp.int32) + cbase
        pltpu.sync_copy(sel.at[pl.ds(sid * ro_slots, ro_slots)],
                        selv.at[pl.ds(0, ro_slots)])
        def ro_pre(g, _):
            edata[pl.ds(g * L, L)] = to16
            edata[pl.ds(lchunk + g * L, L)] = tr16
            return 0
        lax.fori_loop(0, ro_slots // L, ro_pre, 0)
        jb = sid * ro_slots
        def ro_scan(g, o):
            nd = selv[pl.ds(g * L, L)]
            inr = (nd >= base) & (nd < end)
            jv = jb + g * L + iota
            lv = plsc.load_gather(mask, [jnp.maximum(nd, 0)]) - 1 + cbase
            plsc.store_compressed(edata.at[pl.ds(o, L)], jv, mask=inr)
            plsc.store_compressed(edata.at[pl.ds(lchunk + o, L)], lv,
                                  mask=inr)
            return o + jnp.max(plsc.all_reduce_population_count(inr))
        lax.fori_loop(0, ro_slots // L, ro_scan, jnp.int32(0))

        for h in range(nh):
            for g in range(rh // L):
                jl[pl.ds(g * L, L)] = edata[pl.ds(h * rh + g * L, L)]
                ll[pl.ds(g * L, L)] = edata[pl.ds(lchunk + h * rh + g * L, L)]
            pltpu.async_copy(acc.at[ll], orows, sem2).wait()
            def ro_row(rr, _):
                dv = orows[rr, pl.ds(f, L)]
                dens = one16f * dv[0]
                inv = jnp.where(dens == 0.0, 0.0, 1.0 / dens)
                for j in range(f // L):
                    orows[rr, pl.ds(j * L, L)] = \
                        orows[rr, pl.ds(j * L, L)] * inv
                return 0
            lax.fori_loop(0, rh, ro_row, 0)
            pltpu.sync_copy(orows, out.at[jl])

    mesh = plsc.VectorSubcoreMesh(core_axis_name="c", subcore_axis_name="s")
    kern = pl.kernel(
        body,
        out_type=jax.ShapeDtypeStruct((out_rows, fp), jnp.float32),
        mesh=mesh,
        compiler_params=pltpu.CompilerParams(needs_layout_passes=False),
        scratch_types=[
            pltpu.HBM((NC * sc_rows, fp), jnp.float32),
            pltpu.VMEM_SHARED((NS * ebuf,), jnp.int32),
            pltpu.VMEM((maskn,), jnp.int32),
            pltpu.VMEM((sel_pad,), jnp.int32),
            pltpu.VMEM((lchunk,), jnp.int32),
            pltpu.VMEM((lchunk,), jnp.float32),
            pltpu.VMEM((ebuf,), jnp.int32),
            pltpu.VMEM((2 * ebuf,), jnp.int32),
            pltpu.VMEM((lchunk + b + 3 * L,), jnp.int32),
            pltpu.VMEM((lchunk + b + 3 * L,), jnp.int32),
            pltpu.VMEM((lchunk + b + 3 * L,), jnp.float32),
            pltpu.VMEM((b,), jnp.int32),
            pltpu.VMEM((b, f), jnp.float32),
            pltpu.VMEM((spt, fp), jnp.float32),
            pltpu.VMEM((rh,), jnp.int32),
            pltpu.VMEM((rh,), jnp.int32),
            pltpu.VMEM((rh, fp), jnp.float32),
            pltpu.SemaphoreType.DMA,
            pltpu.SemaphoreType.DMA,
        ],
    )
    return kern, e_pad, sel_pad


def kernel(features, weights_down, nidx_down, sel_idx_up):
    n, f = features.shape
    k = weights_down.shape[1]
    n_up = sel_idx_up.shape[0]
    kern, e_pad, sel_pad = _build(n, k, f, n_up)
    e_total = n * k
    nidxf = jnp.concatenate(
        [nidx_down.reshape(-1),
         jnp.full((e_pad - e_total,), n, jnp.int32)])
    wf = jnp.concatenate(
        [weights_down.reshape(-1),
         jnp.zeros((e_pad - e_total,), jnp.float32)])
    selp = jnp.concatenate(
        [sel_idx_up[:, 0].astype(jnp.int32),
         jnp.full((sel_pad - n_up,), -1, jnp.int32)])
    out = kern(features, nidxf, wf, selp)
    return out[:n_up, :f]


# speedup vs baseline: 14.4062x; 1.0878x over previous
import functools

import jax
import jax.numpy as jnp
from jax import lax
from jax.experimental import pallas as pl
from jax.experimental.pallas import tpu as pltpu
from jax.experimental.pallas import tpu_sc as plsc

L = 16
NC = 2
NS = 16


@functools.lru_cache(maxsize=None)
def _build(n, k, f, n_up):
    fp = f + 128
    fpt = f + L
    e_total = n * k
    lchunk = 2048
    rounds = -(-e_total // (NS * lchunk))
    ep_tile = rounds * lchunk
    e_pad = NS * ep_tile
    half = (n // 2 + L - 1) // L * L
    maskn = (n + 1 + L - 1) // L * L
    b = 64
    minq = 4 * b
    rh = 32
    nh = 6
    ro_slots = nh * rh
    sel_pad = NS * ro_slots
    out_rows = sel_pad + L
    out_trash = out_rows - 1
    spt = (sel_pad // NS + 15) // 8 * 8
    sc_rows = NS * spt
    sbits = 14
    ebuf = (L + 2 * lchunk + 127) // 128 * 128
    bstr = 2 * ebuf
    osz = lchunk + minq + 4 * L

    def body(feat, nidxf, wf, sel, out,
             acc, lsh,
             mask, selv, nidx_c, w_c, edata, bdata,
             odst, osrc, ow, srcbuf, gbuf, accT, jl, ll, orows,
             sem, sem2, psem):
        cid = lax.axis_index("c")
        sid = lax.axis_index("s")
        base = cid * half
        end = base + jnp.where(cid == 0, half, n - half)
        cbase = cid * sc_rows
        own0 = sid * spt
        iota = lax.iota(jnp.int32, L)
        zero16 = jnp.zeros((L,), jnp.float32)
        e0vec = jnp.where(iota == 0, 1.0, 0.0).astype(jnp.float32)
        sh5 = jnp.full((L,), 5, jnp.int32)
        shs = jnp.full((L,), sbits, jnp.int32)
        smask = jnp.full((L,), (1 << sbits) - 1, jnp.int32)
        one16f = jnp.full((L,), 1.0, jnp.float32)
        one16i = jnp.full((L,), 1, jnp.int32)

        def zrow(r, _):
            for j in range(fp // L):
                accT[r, pl.ds(j * L, L)] = zero16
            return 0
        lax.fori_loop(0, spt, zrow, 0)

        def zm(g, _):
            mask[pl.ds(g * L, L)] = jnp.zeros((L,), jnp.int32)
            return 0
        lax.fori_loop(0, maskn // L, zm, 0)
        pltpu.sync_copy(sel, selv)
        def bm(g, _):
            nd = selv[pl.ds(g * L, L)]
            inr = (nd >= base) & (nd < end)
            jval = g * L + iota + 1
            plsc.store_scatter(mask, [nd], jval, mask=inr)
            return 0
        lax.fori_loop(0, sel_pad // L, bm, 0)

        def fire_gather(bi):
            slot = lax.rem(bi, 2)
            for q in range(b // L):
                srcbuf[pl.ds(slot * 128 + q * L, L)] = \
                    osrc[pl.ds(bi * b + q * L, L)]
            pltpu.make_async_copy(
                feat.at[srcbuf.at[pl.ds(slot * 128, b)]],
                gbuf.at[pl.ds(slot * b, b)], sem).start()

        def run_batches(off, minoff):
            nb = jnp.where(off >= minoff, lax.shift_right_logical(off, 6), 0)
            @pl.when(nb > 0)
            def _prime():
                fire_gather(0)
            def batch_body(bi, _):
                slot = lax.rem(bi, 2)
                pltpu.make_async_copy(
                    feat.at[srcbuf.at[pl.ds(slot * 128, b)]],
                    gbuf.at[pl.ds(slot * b, b)], sem).wait()
                @pl.when(bi + 1 < nb)
                def _next():
                    fire_gather(bi + 1)
                gb = slot * b
                bb = bi * b
                def edge_body(e, _):
                    row = odst[pl.ds(bb + e, L)][0]
                    ws = one16f * ow[pl.ds(bb + e, L)][0]
                    for j in range(f // L):
                        plsc.addupdate(accT.at[row, pl.ds(j * L, L)],
                                       gbuf[gb + e, pl.ds(j * L, L)] * ws)
                    plsc.addupdate(accT.at[row, pl.ds(f, L)], ws * e0vec)
                    return 0
                lax.fori_loop(0, b, edge_body, 0)
                return 0
            lax.fori_loop(0, nb, batch_body, 0)
            tb = nb * b
            @pl.when(nb > 0)
            def _move_tail():
                for t in range(b // L):
                    odst[pl.ds(t * L, L)] = odst[pl.ds(tb + t * L, L)]
                    osrc[pl.ds(t * L, L)] = osrc[pl.ds(tb + t * L, L)]
                    ow[pl.ds(t * L, L)] = ow[pl.ds(tb + t * L, L)]
            return off - tb

        def fire_pull(t):
            slot = lax.rem(t, 2)
            pltpu.make_async_copy(
                lsh.at[pl.ds(t * 2 * ebuf, 2 * ebuf)],
                bdata.at[pl.ds(slot * bstr, 2 * ebuf)], psem).start()

        def round_body(r, off):
            e0 = sid * ep_tile + r * lchunk
            pltpu.sync_copy(nidxf.at[pl.ds(e0, lchunk)], nidx_c)
            pltpu.sync_copy(wf.at[pl.ds(e0, lchunk)], w_c)
            def scan_g(g, sc_off):
                nd = nidx_c[pl.ds(g * L, L)]
                wv = w_c[pl.ds(g * L, L)]
                m = plsc.load_gather(mask, [nd])
                keep = m > 0
                src = lax.shift_right_logical(e0 + g * L + iota, sh5)
                packed = lax.shift_left(m - 1, shs) | src
                plsc.store_compressed(edata.at[pl.ds(L + sc_off, L)], packed,
                                      mask=keep)
                plsc.store_compressed(edata.at[pl.ds(L + lchunk + sc_off, L)],
                                      plsc.bitcast(wv, jnp.int32), mask=keep)
                return sc_off + jnp.max(plsc.all_reduce_population_count(keep))
            cnt = lax.fori_loop(0, lchunk // L, scan_g, jnp.int32(0))
            edata[pl.ds(0, L)] = one16i * cnt
            pltpu.sync_copy(edata, lsh.at[pl.ds(sid * ebuf, ebuf)])
            plsc.subcore_barrier()

            fire_pull(0)
            def per_pair(t, off):
                slot = lax.rem(t, 2)
                pltpu.make_async_copy(
                    lsh.at[pl.ds(t * 2 * ebuf, 2 * ebuf)],
                    bdata.at[pl.ds(slot * bstr, 2 * ebuf)], psem).wait()
                @pl.when(t + 1 < NS // 2)
                def _next():
                    fire_pull(t + 1)
                sb = slot * bstr
                for sub in (0, ebuf):
                    cnt_t = bdata[pl.ds(sb + sub, L)][0]
                    cnts = one16i * cnt_t
                    ng = lax.shift_right_logical(cnt_t + (L - 1), 4)
                    def own_g(g, o):
                        pk = bdata[pl.ds(sb + sub + L + g * L, L)]
                        s = lax.shift_right_logical(pk, shs)
                        valid = ((g * L + iota) < cnts) & (s >= own0) \
                            & (s < own0 + spt)
                        plsc.store_compressed(odst.at[pl.ds(o, L)], s - own0,
                                              mask=valid)
                        plsc.store_compressed(osrc.at[pl.ds(o, L)],
                                              pk & smask, mask=valid)
                        plsc.store_compressed(
                            ow.at[pl.ds(o, L)],
                            plsc.bitcast(
                                bdata[pl.ds(sb + sub + L + lchunk + g * L, L)],
                                jnp.float32), mask=valid)
                        return o + jnp.max(
                            plsc.all_reduce_population_count(valid))
                    off = lax.fori_loop(0, ng, own_g, off)
                    off = run_batches(off, minq)
                return off
            off = lax.fori_loop(0, NS // 2, per_pair, off)
            plsc.subcore_barrier()
            return off
        off = lax.fori_loop(0, rounds, round_body, jnp.int32(0))

        a16 = off & jnp.int32(~(L - 1))
        offs = one16i * off
        zi = jnp.zeros((L,), jnp.int32)
        zf = jnp.zeros((L,), jnp.float32)
        for t in range(b // L + 1):
            p = a16 + t * L
            mg = (p + iota) >= offs
            odst[pl.ds(p, L)] = jnp.where(mg, zi, odst[pl.ds(p, L)])
            osrc[pl.ds(p, L)] = jnp.where(mg, zi, osrc[pl.ds(p, L)])
            ow[pl.ds(p, L)] = jnp.where(mg, zf, ow[pl.ds(p, L)])
        run_batches((off + b - 1) & jnp.int32(~(b - 1)), 1)

        pltpu.sync_copy(accT, acc.at[pl.ds(cbase + own0, spt)])
        plsc.subcore_barrier()

        to16 = jnp.full((L,), out_trash, jnp.int32)
        tr16 = jnp.full((L,), sc_rows - 1, jnp.int32) + cbase
        pltpu.sync_copy(sel.at[pl.ds(sid * ro_slots, ro_slots)],
                        selv.at[pl.ds(0, ro_slots)])
        def ro_pre(g, _):
            edata[pl.ds(g * L, L)] = to16
            edata[pl.ds(lchunk + g * L, L)] = tr16
            return 0
        lax.fori_loop(0, ro_slots // L, ro_pre, 0)
        jb = sid * ro_slots
        def ro_scan(g, o):
            nd = selv[pl.ds(g * L, L)]
            inr = (nd >= base) & (nd < end)
            jv = jb + g * L + iota
            lv = plsc.load_gather(mask, [jnp.maximum(nd, 0)]) - 1 + cbase
            plsc.store_compressed(edata.at[pl.ds(o, L)], jv, mask=inr)
            plsc.store_compressed(edata.at[pl.ds(lchunk + o, L)], lv,
                                  mask=inr)
            return o + jnp.max(plsc.all_reduce_population_count(inr))
        lax.fori_loop(0, ro_slots // L, ro_scan, jnp.int32(0))

        for h in range(nh):
            for g in range(rh // L):
                jl[pl.ds(g * L, L)] = edata[pl.ds(h * rh + g * L, L)]
                ll[pl.ds(g * L, L)] = edata[pl.ds(lchunk + h * rh + g * L, L)]
            pltpu.async_copy(acc.at[ll], orows, sem2).wait()
            def ro_row(rr, _):
                dv = orows[rr, pl.ds(f, L)]
                dens = one16f * dv[0]
                inv = jnp.where(dens == 0.0, 0.0, 1.0 / dens)
                for j in range(f // L):
                    orows[rr, pl.ds(j * L, L)] = \
                        orows[rr, pl.ds(j * L, L)] * inv
                return 0
            lax.fori_loop(0, rh, ro_row, 0)
            pltpu.sync_copy(orows, out.at[jl])

    mesh = plsc.VectorSubcoreMesh(core_axis_name="c", subcore_axis_name="s")
    kern = pl.kernel(
        body,
        out_type=jax.ShapeDtypeStruct((out_rows, fp), jnp.float32),
        mesh=mesh,
        compiler_params=pltpu.CompilerParams(needs_layout_passes=False),
        scratch_types=[
            pltpu.HBM((NC * sc_rows, fp), jnp.float32),
            pltpu.VMEM_SHARED((NS * ebuf,), jnp.int32),
            pltpu.VMEM((maskn,), jnp.int32),
            pltpu.VMEM((sel_pad,), jnp.int32),
            pltpu.VMEM((lchunk,), jnp.int32),
            pltpu.VMEM((lchunk,), jnp.float32),
            pltpu.VMEM((ebuf,), jnp.int32),
            pltpu.VMEM((2 * 2 * ebuf,), jnp.int32),
            pltpu.VMEM((lchunk + minq + 4 * L,), jnp.int32),
            pltpu.VMEM((lchunk + minq + 4 * L,), jnp.int32),
            pltpu.VMEM((lchunk + minq + 4 * L,), jnp.float32),
            pltpu.VMEM((2 * 128,), jnp.int32),
            pltpu.VMEM((2 * b, f), jnp.float32),
            pltpu.VMEM((spt, fp), jnp.float32),
            pltpu.VMEM((rh,), jnp.int32),
            pltpu.VMEM((rh,), jnp.int32),
            pltpu.VMEM((rh, fp), jnp.float32),
            pltpu.SemaphoreType.DMA,
            pltpu.SemaphoreType.DMA,
            pltpu.SemaphoreType.DMA,
        ],
    )
    return kern, e_pad, sel_pad


def kernel(features, weights_down, nidx_down, sel_idx_up):
    n, f = features.shape
    k = weights_down.shape[1]
    n_up = sel_idx_up.shape[0]
    kern, e_pad, sel_pad = _build(n, k, f, n_up)
    e_total = n * k
    nidxf = jnp.concatenate(
        [nidx_down.reshape(-1),
         jnp.full((e_pad - e_total,), n, jnp.int32)])
    wf = jnp.concatenate(
        [weights_down.reshape(-1),
         jnp.zeros((e_pad - e_total,), jnp.float32)])
    selp = jnp.concatenate(
        [sel_idx_up[:, 0].astype(jnp.int32),
         jnp.full((sel_pad - n_up,), -1, jnp.int32)])
    out = kern(features, nidxf, wf, selp)
    return out[:n_up, :f]
